# Initial kernel scaffold; baseline (speedup 1.0000x reference)
import jax
import jax.numpy as jnp
from jax.experimental import pallas as pl


def kernel(senders, receivers, node_features, edge_features, We1, be1, We2, be2, We3, be3, ge, bte, Wn1, bn1, Wn2, bn2, Wn3, bn3, gn, btn):
    raise NotImplementedError("write your pallas kernel here")


# speedup vs baseline: 3.0344x; 3.0344x over previous
import functools

import jax
import jax.numpy as jnp
from jax import lax
from jax.experimental import pallas as pl
from jax.experimental.pallas import tpu as pltpu
from jax.experimental.pallas import tpu_sc as plsc

N = 10000
E = 320000
H = 128

NC = 2
NS = 16
NW = NC * NS
EPW = E // NW
CH = 80
NCHUNK = EPW // CH
NP = 10240
RPT = NP // NS

_mesh = plsc.VectorSubcoreMesh(core_axis_name="c", subcore_axis_name="s")


@functools.partial(
    pl.kernel,
    mesh=_mesh,
    out_type=(jax.ShapeDtypeStruct((E, H), jnp.float32),
              jax.ShapeDtypeStruct((E, H), jnp.float32)),
    scratch_types=[
        pltpu.VMEM((CH,), jnp.int32),
        pltpu.VMEM((CH,), jnp.int32),
        pltpu.VMEM((CH, H), jnp.float32),
        pltpu.VMEM((CH, H), jnp.float32),
        pltpu.SemaphoreType.DMA,
        pltpu.SemaphoreType.DMA,
    ],
)
def _sc_gather(nf, snd, rcv, sf_out, rf_out,
               idx_s, idx_r, rows_s, rows_r, sem_s, sem_r):
    w = lax.axis_index("s") * NC + lax.axis_index("c")
    base0 = w * EPW

    def body(j, carry):
        base = base0 + j * CH
        pltpu.sync_copy(snd.at[pl.ds(base, CH)], idx_s)
        pltpu.sync_copy(rcv.at[pl.ds(base, CH)], idx_r)
        c1 = pltpu.async_copy(nf.at[idx_s], rows_s, sem_s)
        c2 = pltpu.async_copy(nf.at[idx_r], rows_r, sem_r)
        c1.wait()
        c2.wait()
        pltpu.sync_copy(rows_s, sf_out.at[pl.ds(base, CH)])
        pltpu.sync_copy(rows_r, rf_out.at[pl.ds(base, CH)])
        return carry

    lax.fori_loop(0, NCHUNK, body, 0)


@functools.partial(
    pl.kernel,
    mesh=_mesh,
    out_type=jax.ShapeDtypeStruct((NC, NP, H), jnp.float32),
    scratch_types=[
        pltpu.VMEM((CH,), jnp.int32),
        pltpu.VMEM((CH, H), jnp.float32),
        pltpu.VMEM_SHARED((NP, H), jnp.float32),
    ],
)
def _sc_scatter(mlp, rcv, zeros, acc_out, idx_v, rows_v, shared):
    c = lax.axis_index("c")
    s = lax.axis_index("s")
    pltpu.sync_copy(zeros.at[pl.ds(s * RPT, RPT)],
                    shared.at[pl.ds(s * RPT, RPT)])
    plsc.subcore_barrier()
    base0 = c * (E // NC) + s * EPW

    def body(j, carry):
        base = base0 + j * CH
        pltpu.sync_copy(rcv.at[pl.ds(base, CH)], idx_v)
        pltpu.sync_copy(mlp.at[pl.ds(base, CH)], rows_v)
        pltpu.sync_copy(rows_v, shared.at[idx_v], add=True)
        return carry

    lax.fori_loop(0, NCHUNK, body, 0)
    plsc.subcore_barrier()
    pltpu.sync_copy(shared.at[pl.ds(s * RPT, RPT)],
                    acc_out.at[c].at[pl.ds(s * RPT, RPT)])


def _edge_body(sf, rf, ef, w1s, w1r, w1e, b1, w2, b2, w3, b3, g, bt,
               mlp_o, ne_o):
    x = (jnp.dot(sf[...], w1s[...], preferred_element_type=jnp.float32)
         + jnp.dot(rf[...], w1r[...], preferred_element_type=jnp.float32)
         + jnp.dot(ef[...], w1e[...], preferred_element_type=jnp.float32)
         + b1[...])
    h = jnp.maximum(x, 0.0)
    h = jnp.maximum(
        jnp.dot(h, w2[...], preferred_element_type=jnp.float32) + b2[...], 0.0)
    h = jnp.dot(h, w3[...], preferred_element_type=jnp.float32) + b3[...]
    mu = jnp.mean(h, axis=-1, keepdims=True)
    d = h - mu
    var = jnp.mean(d * d, axis=-1, keepdims=True)
    o = d * lax.rsqrt(var + 1e-5) * g[...] + bt[...]
    mlp_o[...] = o
    ne_o[...] = o + ef[...]


def _node_body(nf, acc, w1a, w1b, b1, w2, b2, w3, b3, g, bt, out):
    a = acc[0] + acc[1]
    x = (jnp.dot(nf[...], w1a[...], preferred_element_type=jnp.float32)
         + jnp.dot(a, w1b[...], preferred_element_type=jnp.float32)
         + b1[...])
    h = jnp.maximum(x, 0.0)
    h = jnp.maximum(
        jnp.dot(h, w2[...], preferred_element_type=jnp.float32) + b2[...], 0.0)
    h = jnp.dot(h, w3[...], preferred_element_type=jnp.float32) + b3[...]
    mu = jnp.mean(h, axis=-1, keepdims=True)
    d = h - mu
    var = jnp.mean(d * d, axis=-1, keepdims=True)
    o = d * lax.rsqrt(var + 1e-5) * g[...] + bt[...]
    out[...] = o + nf[...]


BE = 2000
BN = 1000


def _full(shape):
    return pl.BlockSpec(shape, lambda i: (0,) * len(shape))


def _edge_mlp(sf, rf, ef, w1s, w1r, w1e, b1, w2, b2, w3, b3, g, bt):
    row = pl.BlockSpec((BE, H), lambda i: (i, 0))
    wspec = _full((H, H))
    vspec = _full((1, H))
    return pl.pallas_call(
        _edge_body,
        grid=(E // BE,),
        in_specs=[row, row, row, wspec, wspec, wspec, vspec, wspec, vspec,
                  wspec, vspec, vspec, vspec],
        out_specs=[row, row],
        out_shape=(jax.ShapeDtypeStruct((E, H), jnp.float32),
                   jax.ShapeDtypeStruct((E, H), jnp.float32)),
    )(sf, rf, ef, w1s, w1r, w1e, b1, w2, b2, w3, b3, g, bt)


def _node_mlp(nf, acc, w1a, w1b, b1, w2, b2, w3, b3, g, bt):
    row = pl.BlockSpec((BN, H), lambda i: (i, 0))
    accspec = pl.BlockSpec((NC, BN, H), lambda i: (0, i, 0))
    wspec = _full((H, H))
    vspec = _full((1, H))
    return pl.pallas_call(
        _node_body,
        grid=(N // BN,),
        in_specs=[row, accspec, wspec, wspec, vspec, wspec, vspec, wspec,
                  vspec, vspec, vspec],
        out_specs=row,
        out_shape=jax.ShapeDtypeStruct((N, H), jnp.float32),
    )(nf, acc, w1a, w1b, b1, w2, b2, w3, b3, g, bt)


def kernel(senders, receivers, node_features, edge_features,
           We1, be1, We2, be2, We3, be3, ge, bte,
           Wn1, bn1, Wn2, bn2, Wn3, bn3, gn, btn):
    snd = senders[0].astype(jnp.int32)
    rcv = receivers[0].astype(jnp.int32)
    nf = node_features[0]
    ef = edge_features[0]

    sf, rf = _sc_gather(nf, snd, rcv)

    mlp_out, new_edge = _edge_mlp(
        sf, rf, ef,
        We1[0:H], We1[H:2 * H], We1[2 * H:3 * H], be1.reshape(1, H),
        We2, be2.reshape(1, H), We3, be3.reshape(1, H),
        ge.reshape(1, H), bte.reshape(1, H))

    zeros = jnp.zeros((NP, H), jnp.float32)
    acc2 = _sc_scatter(mlp_out, rcv, zeros)

    new_node = _node_mlp(
        nf, acc2,
        Wn1[0:H], Wn1[H:2 * H], bn1.reshape(1, H),
        Wn2, bn2.reshape(1, H), Wn3, bn3.reshape(1, H),
        gn.reshape(1, H), btn.reshape(1, H))

    return new_node[None], new_edge[None]


# speedup vs baseline: 4.0536x; 1.3359x over previous
import functools

import jax
import jax.numpy as jnp
from jax import lax
from jax.experimental import pallas as pl
from jax.experimental.pallas import tpu as pltpu
from jax.experimental.pallas import tpu_sc as plsc

N = 10000
E = 320000
H = 128

NC = 2
NS = 16
NW = NC * NS
EPW = E // NW
CH = 80
NCHUNK = EPW // CH
NP = 10240
RPT = NP // NS

_mesh = plsc.VectorSubcoreMesh(core_axis_name="c", subcore_axis_name="s")


@functools.partial(
    pl.kernel,
    mesh=_mesh,
    out_type=(jax.ShapeDtypeStruct((E, H), jnp.float32),
              jax.ShapeDtypeStruct((E, H), jnp.float32)),
    scratch_types=[
        pltpu.VMEM((CH,), jnp.int32),
        pltpu.VMEM((CH,), jnp.int32),
        pltpu.VMEM((CH,), jnp.int32),
        pltpu.VMEM((CH,), jnp.int32),
        pltpu.VMEM((CH, H), jnp.float32),
        pltpu.VMEM((CH, H), jnp.float32),
        pltpu.VMEM((CH, H), jnp.float32),
        pltpu.VMEM((CH, H), jnp.float32),
        pltpu.SemaphoreType.DMA,
        pltpu.SemaphoreType.DMA,
        pltpu.SemaphoreType.DMA,
        pltpu.SemaphoreType.DMA,
    ],
)
def _sc_gather(nf, snd, rcv, sf_out, rf_out,
               idx_s0, idx_r0, idx_s1, idx_r1,
               rows_s0, rows_r0, rows_s1, rows_r1,
               sem_s0, sem_r0, sem_s1, sem_r1):
    w = lax.axis_index("s") * NC + lax.axis_index("c")
    base0 = w * EPW
    bufs = ((idx_s0, idx_r0, rows_s0, rows_r0, sem_s0, sem_r0),
            (idx_s1, idx_r1, rows_s1, rows_r1, sem_s1, sem_r1))

    def prefetch(j, b):
        idx_s, idx_r, rows_s, rows_r, sem_s, sem_r = bufs[b]
        base = base0 + j * CH
        pltpu.sync_copy(snd.at[pl.ds(base, CH)], idx_s)
        pltpu.sync_copy(rcv.at[pl.ds(base, CH)], idx_r)
        pltpu.async_copy(nf.at[idx_s], rows_s, sem_s)
        pltpu.async_copy(nf.at[idx_r], rows_r, sem_r)

    def drain(j, b):
        idx_s, idx_r, rows_s, rows_r, sem_s, sem_r = bufs[b]
        base = base0 + j * CH
        pltpu.make_async_copy(nf.at[idx_s], rows_s, sem_s).wait()
        pltpu.make_async_copy(nf.at[idx_r], rows_r, sem_r).wait()
        pltpu.sync_copy(rows_s, sf_out.at[pl.ds(base, CH)])
        pltpu.sync_copy(rows_r, rf_out.at[pl.ds(base, CH)])

    prefetch(0, 0)

    def body(t, carry):
        j0 = 2 * t
        prefetch(j0 + 1, 1)
        drain(j0, 0)
        prefetch(j0 + 2, 0)
        drain(j0 + 1, 1)
        return carry

    lax.fori_loop(0, (NCHUNK - 1) // 2, body, 0)
    drain(NCHUNK - 1, 0)


@functools.partial(
    pl.kernel,
    mesh=_mesh,
    out_type=jax.ShapeDtypeStruct((NC, NP, H), jnp.float32),
    scratch_types=[
        pltpu.VMEM((CH,), jnp.int32),
        pltpu.VMEM((CH,), jnp.int32),
        pltpu.VMEM((CH, H), jnp.float32),
        pltpu.VMEM((CH, H), jnp.float32),
        pltpu.VMEM_SHARED((NP, H), jnp.float32),
        pltpu.SemaphoreType.DMA,
        pltpu.SemaphoreType.DMA,
        pltpu.SemaphoreType.DMA,
        pltpu.SemaphoreType.DMA,
    ],
)
def _sc_scatter(mlp, rcv, zeros, acc_out,
                idx0, idx1, rows0, rows1, shared,
                semi0, semr0, semi1, semr1):
    c = lax.axis_index("c")
    s = lax.axis_index("s")
    pltpu.sync_copy(zeros.at[pl.ds(s * RPT, RPT)],
                    shared.at[pl.ds(s * RPT, RPT)])
    plsc.subcore_barrier()
    base0 = c * (E // NC) + s * EPW
    bufs = ((idx0, rows0, semi0, semr0), (idx1, rows1, semi1, semr1))

    def prefetch(j, b):
        idx_v, rows_v, semi, semr = bufs[b]
        base = base0 + j * CH
        pltpu.async_copy(rcv.at[pl.ds(base, CH)], idx_v, semi)
        pltpu.async_copy(mlp.at[pl.ds(base, CH)], rows_v, semr)

    def drain(j, b):
        idx_v, rows_v, semi, semr = bufs[b]
        base = base0 + j * CH
        pltpu.make_async_copy(rcv.at[pl.ds(base, CH)], idx_v, semi).wait()
        pltpu.make_async_copy(mlp.at[pl.ds(base, CH)], rows_v, semr).wait()
        pltpu.sync_copy(rows_v, shared.at[idx_v], add=True)

    prefetch(0, 0)

    def body(t, carry):
        j0 = 2 * t
        prefetch(j0 + 1, 1)
        drain(j0, 0)
        prefetch(j0 + 2, 0)
        drain(j0 + 1, 1)
        return carry

    lax.fori_loop(0, (NCHUNK - 1) // 2, body, 0)
    drain(NCHUNK - 1, 0)
    plsc.subcore_barrier()
    pltpu.sync_copy(shared.at[pl.ds(s * RPT, RPT)],
                    acc_out.at[c].at[pl.ds(s * RPT, RPT)])


def _edge_body(sf, rf, ef, w1s, w1r, w1e, b1, w2, b2, w3, b3, g, bt,
               mlp_o, ne_o):
    x = (jnp.dot(sf[...], w1s[...], preferred_element_type=jnp.float32)
         + jnp.dot(rf[...], w1r[...], preferred_element_type=jnp.float32)
         + jnp.dot(ef[...], w1e[...], preferred_element_type=jnp.float32)
         + b1[...])
    h = jnp.maximum(x, 0.0)
    h = jnp.maximum(
        jnp.dot(h, w2[...], preferred_element_type=jnp.float32) + b2[...], 0.0)
    h = jnp.dot(h, w3[...], preferred_element_type=jnp.float32) + b3[...]
    mu = jnp.mean(h, axis=-1, keepdims=True)
    d = h - mu
    var = jnp.mean(d * d, axis=-1, keepdims=True)
    o = d * lax.rsqrt(var + 1e-5) * g[...] + bt[...]
    mlp_o[...] = o
    ne_o[...] = o + ef[...]


def _node_body(nf, acc, w1a, w1b, b1, w2, b2, w3, b3, g, bt, out):
    a = acc[0] + acc[1]
    x = (jnp.dot(nf[...], w1a[...], preferred_element_type=jnp.float32)
         + jnp.dot(a, w1b[...], preferred_element_type=jnp.float32)
         + b1[...])
    h = jnp.maximum(x, 0.0)
    h = jnp.maximum(
        jnp.dot(h, w2[...], preferred_element_type=jnp.float32) + b2[...], 0.0)
    h = jnp.dot(h, w3[...], preferred_element_type=jnp.float32) + b3[...]
    mu = jnp.mean(h, axis=-1, keepdims=True)
    d = h - mu
    var = jnp.mean(d * d, axis=-1, keepdims=True)
    o = d * lax.rsqrt(var + 1e-5) * g[...] + bt[...]
    out[...] = o + nf[...]


BE = 2000
BN = 1000


def _full(shape):
    return pl.BlockSpec(shape, lambda i: (0,) * len(shape))


def _edge_mlp(sf, rf, ef, w1s, w1r, w1e, b1, w2, b2, w3, b3, g, bt):
    row = pl.BlockSpec((BE, H), lambda i: (i, 0))
    wspec = _full((H, H))
    vspec = _full((1, H))
    return pl.pallas_call(
        _edge_body,
        grid=(E // BE,),
        in_specs=[row, row, row, wspec, wspec, wspec, vspec, wspec, vspec,
                  wspec, vspec, vspec, vspec],
        out_specs=[row, row],
        out_shape=(jax.ShapeDtypeStruct((E, H), jnp.float32),
                   jax.ShapeDtypeStruct((E, H), jnp.float32)),
    )(sf, rf, ef, w1s, w1r, w1e, b1, w2, b2, w3, b3, g, bt)


def _node_mlp(nf, acc, w1a, w1b, b1, w2, b2, w3, b3, g, bt):
    row = pl.BlockSpec((BN, H), lambda i: (i, 0))
    accspec = pl.BlockSpec((NC, BN, H), lambda i: (0, i, 0))
    wspec = _full((H, H))
    vspec = _full((1, H))
    return pl.pallas_call(
        _node_body,
        grid=(N // BN,),
        in_specs=[row, accspec, wspec, wspec, vspec, wspec, vspec, wspec,
                  vspec, vspec, vspec],
        out_specs=row,
        out_shape=jax.ShapeDtypeStruct((N, H), jnp.float32),
    )(nf, acc, w1a, w1b, b1, w2, b2, w3, b3, g, bt)


def kernel(senders, receivers, node_features, edge_features,
           We1, be1, We2, be2, We3, be3, ge, bte,
           Wn1, bn1, Wn2, bn2, Wn3, bn3, gn, btn):
    snd = senders[0].astype(jnp.int32)
    rcv = receivers[0].astype(jnp.int32)
    nf = node_features[0]
    ef = edge_features[0]

    sf, rf = _sc_gather(nf, snd, rcv)

    mlp_out, new_edge = _edge_mlp(
        sf, rf, ef,
        We1[0:H], We1[H:2 * H], We1[2 * H:3 * H], be1.reshape(1, H),
        We2, be2.reshape(1, H), We3, be3.reshape(1, H),
        ge.reshape(1, H), bte.reshape(1, H))

    zeros = jnp.zeros((NP, H), jnp.float32)
    acc2 = _sc_scatter(mlp_out, rcv, zeros)

    new_node = _node_mlp(
        nf, acc2,
        Wn1[0:H], Wn1[H:2 * H], bn1.reshape(1, H),
        Wn2, bn2.reshape(1, H), Wn3, bn3.reshape(1, H),
        gn.reshape(1, H), btn.reshape(1, H))

    return new_node[None], new_edge[None]
